# Initial kernel scaffold; baseline (speedup 1.0000x reference)
import jax
import jax.numpy as jnp
from jax.experimental import pallas as pl


def kernel(latent_vectors, adjacency_matrix, W_gnn1, b_gnn1, W_gnn2, b_gnn2, Wg1, bg1, gamma1, beta1, Wg2, bg2, gamma2, beta2):
    raise NotImplementedError("write your pallas kernel here")


# speedup vs baseline: 1805.6427x; 1805.6427x over previous
import jax
import jax.numpy as jnp
from jax.experimental import pallas as pl

_N = 1024
_LATENT = 64
_HIDDEN = 256
_OUT = 2000
_NUM_OMICS = 3
_EPS = 1e-3
_ROWS = 8


def _moum_kernel(at_ref, x_ref, w1_ref, b1_ref, w2_ref, b2_ref,
                 wg1_ref, bg1_ref, g1_ref, be1_ref,
                 wg2_ref, bg2_ref, g2_ref, be2_ref, out_ref):
    at = at_ref[...].astype(jnp.float32)
    deg = jnp.sum(at, axis=1, keepdims=True) + 1.0
    norm = jax.lax.rsqrt(jnp.maximum(deg, 1.0))

    x = x_ref[...]
    y = x * norm
    z = jnp.dot(at, y, preferred_element_type=jnp.float32) + y
    agg = z * norm
    x1 = jnp.maximum(
        jnp.dot(agg, w1_ref[...], preferred_element_type=jnp.float32) + b1_ref[...],
        0.0)

    y1 = x1 * norm
    z2 = jnp.dot(at[0:_ROWS, :], y1, preferred_element_type=jnp.float32) + y1[0:_ROWS, :]
    agg2 = z2 * norm[0:_ROWS, :]
    x2 = jnp.maximum(
        jnp.dot(agg2, w2_ref[...], preferred_element_type=jnp.float32) + b2_ref[...],
        0.0)

    inv = 1.0 / jnp.sqrt(1.0 + _EPS)
    rows = []
    for i in range(_NUM_OMICS):
        xi = x2[i:i + 1, :]
        h = jnp.dot(xi, wg1_ref[i], preferred_element_type=jnp.float32) + bg1_ref[i:i + 1, :]
        h = g1_ref[i:i + 1, :] * h * inv + be1_ref[i:i + 1, :]
        h = jnp.maximum(h, 0.0)
        o = jnp.dot(h, wg2_ref[i], preferred_element_type=jnp.float32) + bg2_ref[i:i + 1, :]
        o = g2_ref[i:i + 1, :] * o * inv + be2_ref[i:i + 1, :]
        rows.append(o)
    out_ref[...] = jnp.concatenate(rows, axis=0)


def kernel(latent_vectors, adjacency_matrix, W_gnn1, b_gnn1, W_gnn2, b_gnn2,
           Wg1, bg1, gamma1, beta1, Wg2, bg2, gamma2, beta2):
    at = adjacency_matrix.T.astype(jnp.int8)
    return pl.pallas_call(
        _moum_kernel,
        out_shape=jax.ShapeDtypeStruct((_NUM_OMICS, _OUT), jnp.float32),
    )(at, latent_vectors,
      W_gnn1, b_gnn1.reshape(1, _LATENT), W_gnn2, b_gnn2.reshape(1, _LATENT),
      Wg1, bg1, gamma1, beta1, Wg2, bg2, gamma2, beta2)
